# baseline (speedup 1.0000x reference)
import functools

import numpy as np
import jax
import jax.numpy as jnp
from jax import lax
from jax.experimental import pallas as pl
from jax.experimental.pallas import tpu as pltpu

_N_E = 8192
_E_DIM = 256
_BETA = 0.25
_ALPHA = 0.5
_V_PATCH = (1, 2, 3, 4, 5, 6, 8, 16)
_B, _H, _W = 16, 16, 16
_HW = _H * _W
_NS = len(_V_PATCH)
_NTOK = _B * _HW
_NB = 512
_PAD = 24


def _cubic(x, a=-0.75):
    x = abs(x)
    if x <= 1.0:
        return (a + 2.0) * x ** 3 - (a + 3.0) * x ** 2 + 1.0
    if x < 2.0:
        return a * x ** 3 - 5.0 * a * x ** 2 + 8.0 * a * x - 4.0 * a
    return 0.0


def _bicubic_mat(in_size, out_size):
    M = np.zeros((out_size, in_size), dtype=np.float64)
    scale = in_size / out_size
    for i in range(out_size):
        src = (i + 0.5) * scale - 0.5
        f = int(np.floor(src))
        t = src - f
        for k in range(-1, 3):
            idx = min(max(f + k, 0), in_size - 1)
            M[i, idx] += _cubic(k - t)
    return jnp.asarray(M, dtype=jnp.float32)


def _pool_mat(in_size, out_size):
    M = np.zeros((out_size, in_size), dtype=np.float64)
    for i in range(out_size):
        s = (i * in_size) // out_size
        e = -(((-(i + 1)) * in_size) // out_size)
        M[i, s:e] = 1.0 / (e - s)
    return jnp.asarray(M, dtype=jnp.float32)


_POOLS = {pn: (_pool_mat(_H, pn), _pool_mat(_W, pn)) for pn in _V_PATCH[:-1]}
_UPS = {pn: (_bicubic_mat(pn, _H), _bicubic_mat(pn, _W)) for pn in _V_PATCH[:-1]}


def _argmin_body(zd_ref, emb_ref, rowsq_ref, colsq_ref, idx_ref, best_ref, bidx_ref):
    j = pl.program_id(0)
    t = zd_ref.shape[0]
    zd = zd_ref[...].astype(jnp.bfloat16)
    emb = emb_ref[...].astype(jnp.bfloat16)
    mm = lax.dot_general(zd, emb, (((1,), (1,)), ((), ())),
                         preferred_element_type=jnp.float32)
    dist = (rowsq_ref[...] + colsq_ref[...]) - 2.0 * mm
    lmin = jnp.min(dist, axis=1, keepdims=True)
    liota = lax.broadcasted_iota(jnp.int32, (t, _NB), 1)
    lidx = jnp.min(jnp.where(dist == lmin, liota, _NB), axis=1, keepdims=True) + j * _NB

    @pl.when(j == 0)
    def _():
        best_ref[...] = lmin
        bidx_ref[...] = lidx

    @pl.when(j > 0)
    def _():
        upd = lmin < best_ref[...]
        best_ref[...] = jnp.where(upd, lmin, best_ref[...])
        bidx_ref[...] = jnp.where(upd, lidx, bidx_ref[...])

    @pl.when(j == _N_E // _NB - 1)
    def _():
        idx_ref[...] = bidx_ref[...]


@functools.partial(jax.jit, static_argnames=("t",))
def _argmin_call(zd, emb, rowsq, colsq, t):
    return pl.pallas_call(
        _argmin_body,
        grid=(_N_E // _NB,),
        in_specs=[
            pl.BlockSpec((t, _E_DIM), lambda j: (0, 0)),
            pl.BlockSpec((_NB, _E_DIM), lambda j: (j, 0)),
            pl.BlockSpec((t, 1), lambda j: (0, 0)),
            pl.BlockSpec((1, _NB), lambda j: (0, j)),
        ],
        out_specs=pl.BlockSpec((t, 1), lambda j: (0, 0)),
        out_shape=jax.ShapeDtypeStruct((t, 1), jnp.int32),
        scratch_shapes=[pltpu.VMEM((t, 1), jnp.float32), pltpu.VMEM((t, 1), jnp.int32)],
    )(zd, emb, rowsq, colsq)


def _conv_body(zup_ref, w_ref, b_ref, z_ref, zhat_ref, zrest_ref,
               zhat_out, zrest_out, loss_out, pad_ref):
    pad_ref[0:_PAD, :] = jnp.zeros((_PAD, _E_DIM), jnp.float32)
    pad_ref[_PAD + _NTOK:, :] = jnp.zeros((_PAD, _E_DIM), jnp.float32)
    pad_ref[_PAD:_PAD + _NTOK, :] = zup_ref[...]

    riota = lax.broadcasted_iota(jnp.int32, (_NTOK, 1), 0)
    yy = (riota // _W) % _H
    xx = riota % _W

    acc = None
    for ky in range(3):
        for kx in range(3):
            dy, dx = ky - 1, kx - 1
            s = _PAD + _W * dy + dx
            patch = pad_ref[s:s + _NTOK, :]
            okay = (yy + dy >= 0) & (yy + dy < _H) & (xx + dx >= 0) & (xx + dx < _W)
            patch = jnp.where(okay, patch, 0.0)
            wk = w_ref[(3 * ky + kx) * _E_DIM:(3 * ky + kx + 1) * _E_DIM, :]
            term = lax.dot_general(patch.astype(jnp.bfloat16), wk.astype(jnp.bfloat16),
                                   (((1,), (0,)), ((), ())),
                                   preferred_element_type=jnp.float32)
            acc = term if acc is None else acc + term

    conv_out = acc + b_ref[...]
    resid = zup_ref[...] * (1.0 - _ALPHA) + conv_out * _ALPHA
    zh = zhat_ref[...] + resid
    zhat_out[...] = zh
    zrest_out[...] = zrest_ref[...] - resid
    df = zh - z_ref[...]
    loss_out[...] = jnp.sum(df * df, keepdims=True).reshape(1, 1)


@jax.jit
def _conv_call(zup, w9, bias, z_tok, zhat, zrest):
    return pl.pallas_call(
        _conv_body,
        out_shape=(
            jax.ShapeDtypeStruct((_NTOK, _E_DIM), jnp.float32),
            jax.ShapeDtypeStruct((_NTOK, _E_DIM), jnp.float32),
            jax.ShapeDtypeStruct((1, 1), jnp.float32),
        ),
        scratch_shapes=[pltpu.VMEM((_NTOK + 2 * _PAD, _E_DIM), jnp.float32)],
    )(zup, w9, bias, z_tok, zhat, zrest)


def kernel(z, embedding, Wconv, bconv):
    z_tok = jnp.transpose(z, (0, 2, 3, 1)).reshape(_NTOK, _E_DIM)
    colsq = jnp.sum(embedding ** 2, axis=1).reshape(1, _N_E)
    w9s = jnp.transpose(Wconv, (0, 3, 4, 2, 1)).reshape(_NS, 9 * _E_DIM, _E_DIM)

    zhat = jnp.zeros((_NTOK, _E_DIM), jnp.float32)
    zrest = z_tok
    total_counts = jnp.zeros((_N_E,), dtype=jnp.float32)
    loss_parts = []

    for si, pn in enumerate(_V_PATCH):
        last = si == _NS - 1
        if last:
            zd = zrest
            t = _NTOK
        else:
            Ph, Pw = _POOLS[pn]
            zr4 = jnp.transpose(zrest.reshape(_B, _H, _W, _E_DIM), (0, 3, 1, 2))
            z_down = jnp.einsum('ph,bchw,qw->bcpq', Ph, zr4, Pw)
            zd = jnp.transpose(z_down, (0, 2, 3, 1)).reshape(-1, _E_DIM)
            t = _B * pn * pn
        rowsq = jnp.sum(zd ** 2, axis=1, keepdims=True)
        idx = _argmin_call(zd, embedding, rowsq, colsq, t).reshape(-1)
        z_k = embedding[idx]
        if last:
            zup = z_k
        else:
            Uh, Uw = _UPS[pn]
            zk4 = jnp.transpose(z_k.reshape(_B, pn, pn, _E_DIM), (0, 3, 1, 2))
            z_up4 = jnp.einsum('hp,bcpq,wq->bchw', Uh, zk4, Uw)
            zup = jnp.transpose(z_up4, (0, 2, 3, 1)).reshape(_NTOK, _E_DIM)
        zhat, zrest, lp = _conv_call(zup, w9s[si], bconv[si].reshape(1, _E_DIM),
                                     z_tok, zhat, zrest)
        loss_parts.append(lp.reshape(()))
        total_counts = total_counts + jnp.zeros((_N_E,), jnp.float32).at[idx].add(1.0)

    total_loss = jnp.zeros((), jnp.float32)
    for lp in loss_parts:
        total_loss = total_loss + _BETA * (lp / float(_NTOK * _E_DIM))
    mean_vq_loss = total_loss / _NS

    zh4 = jnp.transpose(zhat.reshape(_B, _H, _W, _E_DIM), (0, 3, 1, 2))
    z_hat_out = z + lax.stop_gradient(zh4 - z)
    return (z_hat_out, mean_vq_loss, total_counts)


# speedup vs baseline: 1.0301x; 1.0301x over previous
import functools

import numpy as np
import jax
import jax.numpy as jnp
from jax import lax
from jax.experimental import pallas as pl
from jax.experimental.pallas import tpu as pltpu

_N_E = 8192
_E_DIM = 256
_BETA = 0.25
_ALPHA = 0.5
_V_PATCH = (1, 2, 3, 4, 5, 6, 8, 16)
_B, _H, _W = 16, 16, 16
_HW = _H * _W
_NS = len(_V_PATCH)
_NTOK = _B * _HW
_NB = 512
_PAD = 24


def _cubic(x, a=-0.75):
    x = abs(x)
    if x <= 1.0:
        return (a + 2.0) * x ** 3 - (a + 3.0) * x ** 2 + 1.0
    if x < 2.0:
        return a * x ** 3 - 5.0 * a * x ** 2 + 8.0 * a * x - 4.0 * a
    return 0.0


def _bicubic_mat(in_size, out_size):
    M = np.zeros((out_size, in_size), dtype=np.float64)
    scale = in_size / out_size
    for i in range(out_size):
        src = (i + 0.5) * scale - 0.5
        f = int(np.floor(src))
        t = src - f
        for k in range(-1, 3):
            idx = min(max(f + k, 0), in_size - 1)
            M[i, idx] += _cubic(k - t)
    return jnp.asarray(M, dtype=jnp.float32)


def _pool_mat(in_size, out_size):
    M = np.zeros((out_size, in_size), dtype=np.float64)
    for i in range(out_size):
        s = (i * in_size) // out_size
        e = -(((-(i + 1)) * in_size) // out_size)
        M[i, s:e] = 1.0 / (e - s)
    return jnp.asarray(M, dtype=jnp.float32)


_POOLS = {pn: (_pool_mat(_H, pn), _pool_mat(_W, pn)) for pn in _V_PATCH[:-1]}
_UPS = {pn: (_bicubic_mat(pn, _H), _bicubic_mat(pn, _W)) for pn in _V_PATCH[:-1]}


def _argmin_body(zd_ref, emb_ref, rowsq_ref, colsq_ref, idx_ref, best_ref, bidx_ref):
    j = pl.program_id(0)
    t = zd_ref.shape[0]
    zd = zd_ref[...].astype(jnp.bfloat16)
    emb = emb_ref[...].astype(jnp.bfloat16)
    mm = lax.dot_general(zd, emb, (((1,), (1,)), ((), ())),
                         preferred_element_type=jnp.float32)
    dist = (rowsq_ref[...] + colsq_ref[...]) - 2.0 * mm
    lmin = jnp.min(dist, axis=1, keepdims=True)
    liota = lax.broadcasted_iota(jnp.int32, (t, _NB), 1)
    lidx = jnp.min(jnp.where(dist == lmin, liota, _NB), axis=1, keepdims=True) + j * _NB

    @pl.when(j == 0)
    def _():
        best_ref[...] = lmin
        bidx_ref[...] = lidx

    @pl.when(j > 0)
    def _():
        upd = lmin < best_ref[...]
        best_ref[...] = jnp.where(upd, lmin, best_ref[...])
        bidx_ref[...] = jnp.where(upd, lidx, bidx_ref[...])

    @pl.when(j == _N_E // _NB - 1)
    def _():
        idx_ref[...] = bidx_ref[...]


@functools.partial(jax.jit, static_argnames=("t",))
def _argmin_call(zd, emb, rowsq, colsq, t):
    return pl.pallas_call(
        _argmin_body,
        grid=(_N_E // _NB,),
        in_specs=[
            pl.BlockSpec((t, _E_DIM), lambda j: (0, 0)),
            pl.BlockSpec((_NB, _E_DIM), lambda j: (j, 0)),
            pl.BlockSpec((t, 1), lambda j: (0, 0)),
            pl.BlockSpec((1, _NB), lambda j: (0, j)),
        ],
        out_specs=pl.BlockSpec((t, 1), lambda j: (0, 0)),
        out_shape=jax.ShapeDtypeStruct((t, 1), jnp.int32),
        scratch_shapes=[pltpu.VMEM((t, 1), jnp.float32), pltpu.VMEM((t, 1), jnp.int32)],
    )(zd, emb, rowsq, colsq)


def _conv_body(zup_ref, w_ref, b_ref, z_ref, zhat_ref, zrest_ref,
               zhat_out, zrest_out, loss_out, pm1, pz0, pp1):
    riota = lax.broadcasted_iota(jnp.int32, (_NTOK, 1), 0)
    xsrc = riota % _W
    zup = zup_ref[...]
    ztop = jnp.zeros((_PAD + 8, _E_DIM), jnp.float32)
    zbot = jnp.zeros((_PAD + _W, _E_DIM), jnp.float32)
    for pad_ref, dx in ((pm1, -1), (pz0, 0), (pp1, 1)):
        pad_ref[0:_PAD + 8, :] = ztop
        pad_ref[_PAD + _NTOK - _W:, :] = zbot
        if dx == 0:
            m = zup
        elif dx == 1:
            m = jnp.where(xsrc == 0, 0.0, zup)
        else:
            m = jnp.where(xsrc == _W - 1, 0.0, zup)
        pad_ref[_PAD - dx:_PAD - dx + _NTOK, :] = m

    ydst = (riota // _W) % _H
    acc = None
    for ky in range(3):
        for kx in range(3):
            dy, dx = ky - 1, kx - 1
            pad_ref = (pm1, pz0, pp1)[dx + 1]
            patch = pad_ref[_PAD + _W * dy:_PAD + _W * dy + _NTOK, :]
            if dy == 1:
                patch = jnp.where(ydst == _H - 1, 0.0, patch)
            elif dy == -1:
                patch = jnp.where(ydst == 0, 0.0, patch)
            wk = w_ref[(3 * ky + kx) * _E_DIM:(3 * ky + kx + 1) * _E_DIM, :]
            term = lax.dot_general(patch.astype(jnp.bfloat16), wk.astype(jnp.bfloat16),
                                   (((1,), (0,)), ((), ())),
                                   preferred_element_type=jnp.float32)
            acc = term if acc is None else acc + term

    conv_out = acc + b_ref[...]
    resid = zup_ref[...] * (1.0 - _ALPHA) + conv_out * _ALPHA
    zh = zhat_ref[...] + resid
    zhat_out[...] = zh
    zrest_out[...] = zrest_ref[...] - resid
    df = zh - z_ref[...]
    loss_out[...] = jnp.sum(df * df, keepdims=True).reshape(1, 1)


@jax.jit
def _conv_call(zup, w9, bias, z_tok, zhat, zrest):
    return pl.pallas_call(
        _conv_body,
        out_shape=(
            jax.ShapeDtypeStruct((_NTOK, _E_DIM), jnp.float32),
            jax.ShapeDtypeStruct((_NTOK, _E_DIM), jnp.float32),
            jax.ShapeDtypeStruct((1, 1), jnp.float32),
        ),
        scratch_shapes=[pltpu.VMEM((_NTOK + 2 * _PAD, _E_DIM), jnp.float32)] * 3,
    )(zup, w9, bias, z_tok, zhat, zrest)


def kernel(z, embedding, Wconv, bconv):
    z_tok = jnp.transpose(z, (0, 2, 3, 1)).reshape(_NTOK, _E_DIM)
    colsq = jnp.sum(embedding ** 2, axis=1).reshape(1, _N_E)
    w9s = jnp.transpose(Wconv, (0, 3, 4, 2, 1)).reshape(_NS, 9 * _E_DIM, _E_DIM)

    zhat = jnp.zeros((_NTOK, _E_DIM), jnp.float32)
    zrest = z_tok
    total_counts = jnp.zeros((_N_E,), dtype=jnp.float32)
    loss_parts = []

    for si, pn in enumerate(_V_PATCH):
        last = si == _NS - 1
        if last:
            zd = zrest
            t = _NTOK
        else:
            Ph, Pw = _POOLS[pn]
            zr4 = jnp.transpose(zrest.reshape(_B, _H, _W, _E_DIM), (0, 3, 1, 2))
            z_down = jnp.einsum('ph,bchw,qw->bcpq', Ph, zr4, Pw)
            zd = jnp.transpose(z_down, (0, 2, 3, 1)).reshape(-1, _E_DIM)
            t = _B * pn * pn
        rowsq = jnp.sum(zd ** 2, axis=1, keepdims=True)
        idx = _argmin_call(zd, embedding, rowsq, colsq, t).reshape(-1)
        z_k = embedding[idx]
        if last:
            zup = z_k
        else:
            Uh, Uw = _UPS[pn]
            zk4 = jnp.transpose(z_k.reshape(_B, pn, pn, _E_DIM), (0, 3, 1, 2))
            z_up4 = jnp.einsum('hp,bcpq,wq->bchw', Uh, zk4, Uw)
            zup = jnp.transpose(z_up4, (0, 2, 3, 1)).reshape(_NTOK, _E_DIM)
        zhat, zrest, lp = _conv_call(zup, w9s[si], bconv[si].reshape(1, _E_DIM),
                                     z_tok, zhat, zrest)
        loss_parts.append(lp.reshape(()))
        total_counts = total_counts + jnp.zeros((_N_E,), jnp.float32).at[idx].add(1.0)

    total_loss = jnp.zeros((), jnp.float32)
    for lp in loss_parts:
        total_loss = total_loss + _BETA * (lp / float(_NTOK * _E_DIM))
    mean_vq_loss = total_loss / _NS

    zh4 = jnp.transpose(zhat.reshape(_B, _H, _W, _E_DIM), (0, 3, 1, 2))
    z_hat_out = z + lax.stop_gradient(zh4 - z)
    return (z_hat_out, mean_vq_loss, total_counts)


# speedup vs baseline: 1.0437x; 1.0131x over previous
import functools

import numpy as np
import jax
import jax.numpy as jnp
from jax import lax
from jax.experimental import pallas as pl
from jax.experimental.pallas import tpu as pltpu

_N_E = 8192
_E_DIM = 256
_BETA = 0.25
_ALPHA = 0.5
_V_PATCH = (1, 2, 3, 4, 5, 6, 8, 16)
_B, _H, _W = 16, 16, 16
_HW = _H * _W
_NS = len(_V_PATCH)
_NTOK = _B * _HW
_NB = 512
_PAD = 24


def _cubic(x, a=-0.75):
    x = abs(x)
    if x <= 1.0:
        return (a + 2.0) * x ** 3 - (a + 3.0) * x ** 2 + 1.0
    if x < 2.0:
        return a * x ** 3 - 5.0 * a * x ** 2 + 8.0 * a * x - 4.0 * a
    return 0.0


def _bicubic_mat(in_size, out_size):
    M = np.zeros((out_size, in_size), dtype=np.float64)
    scale = in_size / out_size
    for i in range(out_size):
        src = (i + 0.5) * scale - 0.5
        f = int(np.floor(src))
        t = src - f
        for k in range(-1, 3):
            idx = min(max(f + k, 0), in_size - 1)
            M[i, idx] += _cubic(k - t)
    return M.astype(np.float32)


def _pool_mat(in_size, out_size):
    M = np.zeros((out_size, in_size), dtype=np.float64)
    for i in range(out_size):
        s = (i * in_size) // out_size
        e = -(((-(i + 1)) * in_size) // out_size)
        M[i, s:e] = 1.0 / (e - s)
    return M.astype(np.float32)


_POOLS = {pn: (_pool_mat(_H, pn), _pool_mat(_W, pn)) for pn in _V_PATCH[:-1]}
_UPS = {pn: (_bicubic_mat(pn, _H), _bicubic_mat(pn, _W)) for pn in _V_PATCH[:-1]}


def _argmin_body(zd_ref, emb_ref, rowsq_ref, colsq_ref, idx_ref, best_ref, bidx_ref):
    j = pl.program_id(0)
    t = zd_ref.shape[0]
    zd = zd_ref[...].astype(jnp.bfloat16)
    emb = emb_ref[...].astype(jnp.bfloat16)
    mm = lax.dot_general(zd, emb, (((1,), (1,)), ((), ())),
                         preferred_element_type=jnp.float32)
    dist = (rowsq_ref[...] + colsq_ref[...]) - 2.0 * mm
    lmin = jnp.min(dist, axis=1, keepdims=True)
    liota = lax.broadcasted_iota(jnp.int32, (t, _NB), 1)
    lidx = jnp.min(jnp.where(dist == lmin, liota, _NB), axis=1, keepdims=True) + j * _NB

    @pl.when(j == 0)
    def _():
        best_ref[...] = lmin
        bidx_ref[...] = lidx

    @pl.when(j > 0)
    def _():
        upd = lmin < best_ref[...]
        best_ref[...] = jnp.where(upd, lmin, best_ref[...])
        bidx_ref[...] = jnp.where(upd, lidx, bidx_ref[...])

    @pl.when(j == _N_E // _NB - 1)
    def _():
        idx_ref[...] = bidx_ref[...]


@functools.partial(jax.jit, static_argnames=("t",))
def _argmin_call(zd, emb, rowsq, colsq, t):
    return pl.pallas_call(
        _argmin_body,
        grid=(_N_E // _NB,),
        in_specs=[
            pl.BlockSpec((t, _E_DIM), lambda j: (0, 0)),
            pl.BlockSpec((_NB, _E_DIM), lambda j: (j, 0)),
            pl.BlockSpec((t, 1), lambda j: (0, 0)),
            pl.BlockSpec((1, _NB), lambda j: (0, j)),
        ],
        out_specs=pl.BlockSpec((t, 1), lambda j: (0, 0)),
        out_shape=jax.ShapeDtypeStruct((t, 1), jnp.int32),
        scratch_shapes=[pltpu.VMEM((t, 1), jnp.float32), pltpu.VMEM((t, 1), jnp.int32)],
    )(zd, emb, rowsq, colsq)


def _conv_body(zup_ref, w_ref, b_ref, z_ref, zhat_ref, zrest_ref,
               zhat_out, zrest_out, loss_out, pm1, pz0, pp1):
    riota = lax.broadcasted_iota(jnp.int32, (_NTOK, 1), 0)
    xsrc = riota % _W
    zup = zup_ref[...]
    ztop = jnp.zeros((_PAD + 8, _E_DIM), jnp.float32)
    zbot = jnp.zeros((_PAD + _W, _E_DIM), jnp.float32)
    for pad_ref, dx in ((pm1, -1), (pz0, 0), (pp1, 1)):
        pad_ref[0:_PAD + 8, :] = ztop
        pad_ref[_PAD + _NTOK - _W:, :] = zbot
        if dx == 0:
            m = zup
        elif dx == 1:
            m = jnp.where(xsrc == 0, 0.0, zup)
        else:
            m = jnp.where(xsrc == _W - 1, 0.0, zup)
        pad_ref[_PAD - dx:_PAD - dx + _NTOK, :] = m

    ydst = (riota // _W) % _H
    acc = None
    for ky in range(3):
        for kx in range(3):
            dy, dx = ky - 1, kx - 1
            pad_ref = (pm1, pz0, pp1)[dx + 1]
            patch = pad_ref[_PAD + _W * dy:_PAD + _W * dy + _NTOK, :]
            if dy == 1:
                patch = jnp.where(ydst == _H - 1, 0.0, patch)
            elif dy == -1:
                patch = jnp.where(ydst == 0, 0.0, patch)
            wk = w_ref[(3 * ky + kx) * _E_DIM:(3 * ky + kx + 1) * _E_DIM, :]
            term = lax.dot_general(patch.astype(jnp.bfloat16), wk.astype(jnp.bfloat16),
                                   (((1,), (0,)), ((), ())),
                                   preferred_element_type=jnp.float32)
            acc = term if acc is None else acc + term

    conv_out = acc + b_ref[...]
    resid = zup_ref[...] * (1.0 - _ALPHA) + conv_out * _ALPHA
    zh = zhat_ref[...] + resid
    zhat_out[...] = zh
    zrest_out[...] = zrest_ref[...] - resid
    df = zh - z_ref[...]
    loss_out[...] = jnp.sum(df * df, keepdims=True).reshape(1, 1)


@jax.jit
def _conv_call(zup, w9, bias, z_tok, zhat, zrest):
    return pl.pallas_call(
        _conv_body,
        out_shape=(
            jax.ShapeDtypeStruct((_NTOK, _E_DIM), jnp.float32),
            jax.ShapeDtypeStruct((_NTOK, _E_DIM), jnp.float32),
            jax.ShapeDtypeStruct((1, 1), jnp.float32),
        ),
        scratch_shapes=[pltpu.VMEM((_NTOK + 2 * _PAD, _E_DIM), jnp.float32)] * 3,
    )(zup, w9, bias, z_tok, zhat, zrest)


def kernel(z, embedding, Wconv, bconv):
    z_tok = jnp.transpose(z, (0, 2, 3, 1)).reshape(_NTOK, _E_DIM)
    colsq = jnp.sum(embedding ** 2, axis=1).reshape(1, _N_E)
    w9s = jnp.transpose(Wconv, (0, 3, 4, 2, 1)).reshape(_NS, 9 * _E_DIM, _E_DIM)

    zhat = jnp.zeros((_NTOK, _E_DIM), jnp.float32)
    zrest = z_tok
    total_counts = jnp.zeros((_N_E,), dtype=jnp.float32)
    loss_parts = []

    for si, pn in enumerate(_V_PATCH):
        last = si == _NS - 1
        if last:
            zd = zrest
            t = _NTOK
        else:
            Ph, Pw = _POOLS[pn]
            zr3 = zrest.reshape(_B, _H, _W, _E_DIM)
            t1 = jnp.einsum('qw,bhwc->bhqc', jnp.asarray(Pw), zr3)
            zd = jnp.einsum('ph,bhqc->bpqc', jnp.asarray(Ph), t1).reshape(-1, _E_DIM)
            t = _B * pn * pn
        rowsq = jnp.sum(zd ** 2, axis=1, keepdims=True)
        idx = _argmin_call(zd, embedding, rowsq, colsq, t).reshape(-1)
        z_k = embedding[idx]
        if last:
            zup = z_k
        else:
            Uh, Uw = _UPS[pn]
            zk3 = z_k.reshape(_B, pn, pn, _E_DIM)
            s1 = jnp.einsum('hp,bpqc->bhqc', jnp.asarray(Uh), zk3)
            zup = jnp.einsum('wq,bhqc->bhwc', jnp.asarray(Uw), s1).reshape(_NTOK, _E_DIM)
        zhat, zrest, lp = _conv_call(zup, w9s[si], bconv[si].reshape(1, _E_DIM),
                                     z_tok, zhat, zrest)
        loss_parts.append(lp.reshape(()))
        total_counts = total_counts + jnp.zeros((_N_E,), jnp.float32).at[idx].add(1.0)

    total_loss = jnp.zeros((), jnp.float32)
    for lp in loss_parts:
        total_loss = total_loss + _BETA * (lp / float(_NTOK * _E_DIM))
    mean_vq_loss = total_loss / _NS

    zh4 = jnp.transpose(zhat.reshape(_B, _H, _W, _E_DIM), (0, 3, 1, 2))
    z_hat_out = z + lax.stop_gradient(zh4 - z)
    return (z_hat_out, mean_vq_loss, total_counts)
